# Initial kernel scaffold; baseline (speedup 1.0000x reference)
import jax
import jax.numpy as jnp
from jax.experimental import pallas as pl


def kernel(x, pos_table):
    raise NotImplementedError("write your pallas kernel here")


# speedup vs baseline: 2.8207x; 2.8207x over previous
import jax
import jax.numpy as jnp
from jax.experimental import pallas as pl


_BLK_S = 512


def _add_kernel(x_ref, pos_ref, o_ref):
    o_ref[...] = x_ref[...] + pos_ref[...]


def kernel(x, pos_table):
    b, s, d = x.shape
    grid = (s // _BLK_S, b)
    return pl.pallas_call(
        _add_kernel,
        grid=grid,
        in_specs=[
            pl.BlockSpec((1, _BLK_S, d), lambda i, j: (j, i, 0)),
            pl.BlockSpec((_BLK_S, d), lambda i, j: (i, 0)),
        ],
        out_specs=pl.BlockSpec((1, _BLK_S, d), lambda i, j: (j, i, 0)),
        out_shape=jax.ShapeDtypeStruct((b, s, d), x.dtype),
    )(x, pos_table[:s])
